# Initial kernel scaffold; baseline (speedup 1.0000x reference)
import jax
import jax.numpy as jnp
from jax.experimental import pallas as pl


def kernel(x, edge_index, state, W1, b1, W2, b2, Ws, bs, Wc, bc):
    raise NotImplementedError("write your pallas kernel here")


# speedup vs baseline: 70.3179x; 70.3179x over previous
import functools

import jax
import jax.numpy as jnp
from jax import lax
from jax.experimental import pallas as pl
from jax.experimental.pallas import tpu as pltpu
from jax.experimental.pallas import tpu_sc as plsc

NC = 2
NS = 16
NW = NC * NS
LANES = 128


def _mesh():
  return plsc.VectorSubcoreMesh(core_axis_name="c", subcore_axis_name="s")


def _stage_of(slc):
  return next(s for s in range(512, 7, -8) if slc % s == 0)


def _deg_kernel(n_pad, rows_pt, rb):
  slc = n_pad // NS
  stage = _stage_of(slc)
  n_stage = slc // stage

  @functools.partial(
      pl.kernel,
      out_type=jax.ShapeDtypeStruct((NC * n_pad,), jnp.float32),
      mesh=_mesh(),
      compiler_params=pltpu.CompilerParams(use_tc_tiling_on_sc=False),
      scratch_types=[
          pltpu.VMEM_SHARED((n_pad,), jnp.float32),
          pltpu.VMEM((rb, LANES), jnp.int32),
          pltpu.VMEM((LANES,), jnp.float32),
          pltpu.VMEM((stage,), jnp.float32),
      ],
  )
  def k(dst_hbm, out_hbm, deg_sp, idx_v, ones_v, stg1):
    c = lax.axis_index("c")
    s = lax.axis_index("s")
    wid = c * NS + s
    r0s = s * slc

    @pl.loop(0, stage // 16)
    def _(i):
      stg1[pl.ds(i * 16, 16)] = jnp.zeros((16,), jnp.float32)

    @pl.loop(0, n_stage)
    def _(i):
      pltpu.sync_copy(stg1, deg_sp.at[pl.ds(r0s + i * stage, stage)])

    for j in range(LANES // 16):
      ones_v[pl.ds(j * 16, 16)] = jnp.full((16,), 1.0, jnp.float32)
    plsc.subcore_barrier()
    row_base = wid * rows_pt

    @pl.loop(0, rows_pt // rb)
    def _(b):
      pltpu.sync_copy(dst_hbm.at[pl.ds(row_base + b * rb, rb)], idx_v)
      for r in range(rb):
        pltpu.sync_copy(ones_v, deg_sp.at[idx_v.at[r]], add=True)

    plsc.subcore_barrier()

    @pl.loop(0, n_stage)
    def _(i):
      pltpu.sync_copy(deg_sp.at[pl.ds(r0s + i * stage, stage)], stg1)
      pltpu.sync_copy(stg1, out_hbm.at[pl.ds(c * n_pad + r0s + i * stage,
                                             stage)])

  return k


def _main_kernel(n_pad, rows_pt, rb):
  slc = n_pad // NS
  stage = _stage_of(slc)
  n_stage = slc // stage

  @functools.partial(
      pl.kernel,
      out_type=(jax.ShapeDtypeStruct((NC * n_pad, 8), jnp.float32),
                jax.ShapeDtypeStruct((NC * n_pad,), jnp.float32)),
      mesh=_mesh(),
      compiler_params=pltpu.CompilerParams(use_tc_tiling_on_sc=False),
      scratch_types=[
          pltpu.VMEM_SHARED((n_pad, 8), jnp.float32),
          pltpu.VMEM_SHARED((n_pad, 8), jnp.float32),
          pltpu.VMEM_SHARED((n_pad,), jnp.float32),
          pltpu.VMEM_SHARED((n_pad,), jnp.float32),
          pltpu.VMEM((rb, LANES), jnp.int32),
          pltpu.VMEM((rb, LANES), jnp.int32),
          pltpu.VMEM((LANES, 8), jnp.float32),
          pltpu.VMEM((LANES,), jnp.float32),
          pltpu.VMEM((stage, 8), jnp.float32),
          pltpu.VMEM((stage,), jnp.float32),
      ],
  )
  def k(src_hbm, dst_hbm, y_hbm, dinv_hbm, z8_hbm,
        s_out, c_out, y_sp, s_sp, c_sp, dinv_sp, sidx, didx, ybuf, cvals,
        stg8, stg1):
    c = lax.axis_index("c")
    s = lax.axis_index("s")
    wid = c * NS + s
    r0s = s * slc

    @pl.loop(0, stage // 16)
    def _(i):
      stg1[pl.ds(i * 16, 16)] = jnp.zeros((16,), jnp.float32)

    pltpu.sync_copy(z8_hbm, stg8)

    @pl.loop(0, n_stage)
    def _(i):
      pltpu.sync_copy(stg8, s_sp.at[pl.ds(r0s + i * stage, stage)])
      pltpu.sync_copy(stg1, c_sp.at[pl.ds(r0s + i * stage, stage)])

    @pl.loop(0, n_stage)
    def _(i):
      pltpu.sync_copy(y_hbm.at[pl.ds(r0s + i * stage, stage)], stg8)
      pltpu.sync_copy(stg8, y_sp.at[pl.ds(r0s + i * stage, stage)])
      pltpu.sync_copy(dinv_hbm.at[pl.ds(r0s + i * stage, stage)], stg1)
      pltpu.sync_copy(stg1, dinv_sp.at[pl.ds(r0s + i * stage, stage)])

    plsc.subcore_barrier()
    row_base = wid * rows_pt

    @pl.loop(0, rows_pt // rb)
    def _(b):
      pltpu.sync_copy(src_hbm.at[pl.ds(row_base + b * rb, rb)], sidx)
      pltpu.sync_copy(dst_hbm.at[pl.ds(row_base + b * rb, rb)], didx)
      for r in range(rb):
        pltpu.sync_copy(y_sp.at[sidx.at[r]], ybuf)
        pltpu.sync_copy(ybuf, s_sp.at[didx.at[r]], add=True)
        pltpu.sync_copy(dinv_sp.at[didx.at[r]], cvals)
        pltpu.sync_copy(cvals, c_sp.at[sidx.at[r]], add=True)

    plsc.subcore_barrier()

    @pl.loop(0, n_stage)
    def _(i):
      pltpu.sync_copy(s_sp.at[pl.ds(r0s + i * stage, stage)], stg8)
      pltpu.sync_copy(stg8, s_out.at[pl.ds(c * n_pad + r0s + i * stage,
                                           stage)])
      pltpu.sync_copy(c_sp.at[pl.ds(r0s + i * stage, stage)], stg1)
      pltpu.sync_copy(stg1, c_out.at[pl.ds(c * n_pad + r0s + i * stage,
                                           stage)])

  return k


def _norm_tc(n_pad, n, blk):
  nb = n_pad // blk

  def body(p0_ref, p1_ref, x_ref, dinv_ref, y_ref):
    i = pl.program_id(0)
    deg = p0_ref[...] + p1_ref[...] + 1.0
    dinv = lax.rsqrt(deg)
    rid = lax.broadcasted_iota(jnp.int32, (blk, 1), 0) + i * blk
    dinv = jnp.where(rid < n, dinv, 0.0)
    dinv_ref[...] = dinv
    y_ref[...] = dinv * x_ref[...]

  return pl.pallas_call(
      body,
      grid=(nb,),
      in_specs=[
          pl.BlockSpec((blk, 1), lambda i: (i, 0)),
          pl.BlockSpec((blk, 1), lambda i: (i, 0)),
          pl.BlockSpec((blk, 8), lambda i: (i, 0)),
      ],
      out_specs=[
          pl.BlockSpec((blk, 1), lambda i: (i, 0)),
          pl.BlockSpec((blk, 8), lambda i: (i, 0)),
      ],
      out_shape=(jax.ShapeDtypeStruct((n_pad, 1), jnp.float32),
                 jax.ShapeDtypeStruct((n_pad, 8), jnp.float32)),
  )


def _fuse_tc(n_pad, n, blk):
  nb = n_pad // blk

  def body(s0_ref, s1_ref, c0_ref, c1_ref, dinv_ref, x_ref,
           w1_ref, b1_ref, w2_ref, b2_ref, st_ref, ws_ref, bs_ref,
           wc1_ref, wc2_ref, bc_ref, out_ref, acc):
    i = pl.program_id(0)

    @pl.when(i == 0)
    def _():
      acc[...] = jnp.zeros_like(acc)

    dv = dinv_ref[...]
    a1 = dv * (s0_ref[...] + s1_ref[...]) + dv * dv * x_ref[...]
    h1 = jnp.maximum(
        jnp.dot(a1, w1_ref[...], preferred_element_type=jnp.float32)
        + b1_ref[...], 0.0)
    w = dv * (c0_ref[...] + c1_ref[...] + dv) * (1.0 / n)
    acc[...] += jnp.sum(w * h1, axis=0, keepdims=True)

    @pl.when(i == nb - 1)
    def _():
      pooled = jnp.dot(acc[...], w2_ref[...],
                       preferred_element_type=jnp.float32) + b2_ref[...]
      sf = jnp.maximum(
          jnp.dot(st_ref[...], ws_ref[...],
                  preferred_element_type=jnp.float32) + bs_ref[...], 0.0)
      out_ref[...] = (
          jnp.dot(pooled, wc1_ref[...], preferred_element_type=jnp.float32)
          + jnp.dot(sf, wc2_ref[...], preferred_element_type=jnp.float32)
          + bc_ref[...])

  full = lambda shape: pl.BlockSpec(shape, lambda i: tuple(0 for _ in shape))
  return pl.pallas_call(
      body,
      grid=(nb,),
      in_specs=[
          pl.BlockSpec((blk, 8), lambda i: (i, 0)),
          pl.BlockSpec((blk, 8), lambda i: (i, 0)),
          pl.BlockSpec((blk, 1), lambda i: (i, 0)),
          pl.BlockSpec((blk, 1), lambda i: (i, 0)),
          pl.BlockSpec((blk, 1), lambda i: (i, 0)),
          pl.BlockSpec((blk, 8), lambda i: (i, 0)),
          full((8, 64)), full((1, 64)),
          full((64, 64)), full((1, 64)),
          full((1, 8)), full((8, 64)), full((1, 64)),
          full((64, 2)), full((64, 2)), full((1, 2)),
      ],
      out_specs=pl.BlockSpec((1, 2), lambda i: (0, 0)),
      out_shape=jax.ShapeDtypeStruct((1, 2), jnp.float32),
      scratch_shapes=[pltpu.VMEM((1, 64), jnp.float32)],
  )


def kernel(x, edge_index, state, W1, b1, W2, b2, Ws, bs, Wc, bc):
  n, _ = x.shape
  e = edge_index.shape[1]
  n_pad = ((n + 1 + LANES - 1) // LANES) * LANES
  rows_pt8 = (e + NW * LANES - 1) // (NW * LANES)
  rows_pt = ((rows_pt8 + 7) // 8) * 8
  e_pad = NW * rows_pt * LANES
  rb = 16 if rows_pt % 16 == 0 else 8

  sent = (n + (jnp.arange(e_pad - e, dtype=jnp.int32) % (n_pad - n)))
  src2d = jnp.concatenate([edge_index[0], sent]).reshape(-1, LANES)
  dst2d = jnp.concatenate([edge_index[1], sent]).reshape(-1, LANES)
  x_pad = jnp.pad(x, ((0, n_pad - n), (0, 0)))
  zeros8 = jnp.zeros((_stage_of(n_pad // NS), 8), jnp.float32)

  degp = _deg_kernel(n_pad, rows_pt, rb)(dst2d).reshape(NC, n_pad)
  p0 = degp[0].reshape(n_pad, 1)
  p1 = degp[1].reshape(n_pad, 1)

  blk = n_pad // 16
  dinv2d, y = _norm_tc(n_pad, n, blk)(p0, p1, x_pad)

  sp, cp = _main_kernel(n_pad, rows_pt, rb)(
      src2d, dst2d, y, dinv2d.reshape(n_pad), zeros8)
  sp = sp.reshape(NC, n_pad, 8)
  cp = cp.reshape(NC, n_pad)

  out = _fuse_tc(n_pad, n, blk)(
      sp[0], sp[1], cp[0].reshape(n_pad, 1), cp[1].reshape(n_pad, 1),
      dinv2d, x_pad,
      W1, b1.reshape(1, -1), W2, b2.reshape(1, -1),
      state, Ws, bs.reshape(1, -1), Wc[:64], Wc[64:], bc.reshape(1, -1))
  return out


# speedup vs baseline: 83.8042x; 1.1918x over previous
import functools

import jax
import jax.numpy as jnp
from jax import lax
from jax.experimental import pallas as pl
from jax.experimental.pallas import tpu as pltpu
from jax.experimental.pallas import tpu_sc as plsc

NC = 2
NS = 16
NW = NC * NS
LANES = 128


def _mesh():
  return plsc.VectorSubcoreMesh(core_axis_name="c", subcore_axis_name="s")


def _stage_of(slc):
  return next(s for s in range(512, 7, -8) if slc % s == 0)


def _deg_kernel(n_pad, rows_pt, rb):
  slc = n_pad // NS
  stage = _stage_of(slc)
  n_stage = slc // stage

  @functools.partial(
      pl.kernel,
      out_type=jax.ShapeDtypeStruct((NC * n_pad,), jnp.float32),
      mesh=_mesh(),
      compiler_params=pltpu.CompilerParams(use_tc_tiling_on_sc=False),
      scratch_types=[
          pltpu.VMEM_SHARED((n_pad,), jnp.float32),
          pltpu.VMEM((rb, LANES), jnp.int32),
          pltpu.VMEM((LANES,), jnp.float32),
          pltpu.VMEM((stage,), jnp.float32),
          pltpu.SemaphoreType.DMA,
          pltpu.SemaphoreType.DMA,
      ],
  )
  def k(dst_hbm, out_hbm, deg_sp, idx_v, ones_v, stg1, gsem, ssem):
    c = lax.axis_index("c")
    s = lax.axis_index("s")
    wid = c * NS + s
    r0s = s * slc

    @pl.loop(0, stage // 16)
    def _(i):
      stg1[pl.ds(i * 16, 16)] = jnp.zeros((16,), jnp.float32)

    @pl.loop(0, n_stage)
    def _(i):
      pltpu.sync_copy(stg1, deg_sp.at[pl.ds(r0s + i * stage, stage)])

    for j in range(LANES // 16):
      ones_v[pl.ds(j * 16, 16)] = jnp.full((16,), 1.0, jnp.float32)
    plsc.subcore_barrier()
    row_base = wid * rows_pt

    @pl.loop(0, rows_pt // rb)
    def _(b):
      pltpu.sync_copy(dst_hbm.at[pl.ds(row_base + b * rb, rb)], idx_v)
      descs = [pltpu.async_copy(ones_v, deg_sp.at[idx_v.at[r]], ssem,
                                add=True) for r in range(rb)]
      for d in descs:
        d.wait()

    plsc.subcore_barrier()

    @pl.loop(0, n_stage)
    def _(i):
      pltpu.sync_copy(deg_sp.at[pl.ds(r0s + i * stage, stage)], stg1)
      pltpu.sync_copy(stg1, out_hbm.at[pl.ds(c * n_pad + r0s + i * stage,
                                             stage)])

  return k


def _main_kernel(n_pad, rows_pt, rb):
  slc = n_pad // NS
  stage = _stage_of(slc)
  n_stage = slc // stage

  @functools.partial(
      pl.kernel,
      out_type=(jax.ShapeDtypeStruct((NC * n_pad, 8), jnp.float32),
                jax.ShapeDtypeStruct((NC * n_pad,), jnp.float32)),
      mesh=_mesh(),
      compiler_params=pltpu.CompilerParams(use_tc_tiling_on_sc=False),
      scratch_types=[
          pltpu.VMEM_SHARED((n_pad, 8), jnp.float32),
          pltpu.VMEM_SHARED((n_pad, 8), jnp.float32),
          pltpu.VMEM_SHARED((n_pad,), jnp.float32),
          pltpu.VMEM_SHARED((n_pad,), jnp.float32),
          pltpu.VMEM((rb, LANES), jnp.int32),
          pltpu.VMEM((rb, LANES), jnp.int32),
          pltpu.VMEM((rb, LANES, 8), jnp.float32),
          pltpu.VMEM((rb, LANES), jnp.float32),
          pltpu.VMEM((stage, 8), jnp.float32),
          pltpu.VMEM((stage,), jnp.float32),
          pltpu.SemaphoreType.DMA,
          pltpu.SemaphoreType.DMA,
          pltpu.SemaphoreType.DMA,
          pltpu.SemaphoreType.DMA,
      ],
  )
  def k(src_hbm, dst_hbm, y_hbm, dinv_hbm, z8_hbm,
        s_out, c_out, y_sp, s_sp, c_sp, dinv_sp, sidx, didx, ybufs, cvals,
        stg8, stg1, gsem, dsem, ssem, csem):
    c = lax.axis_index("c")
    s = lax.axis_index("s")
    wid = c * NS + s
    r0s = s * slc

    @pl.loop(0, stage // 16)
    def _(i):
      stg1[pl.ds(i * 16, 16)] = jnp.zeros((16,), jnp.float32)

    pltpu.sync_copy(z8_hbm, stg8)

    @pl.loop(0, n_stage)
    def _(i):
      pltpu.sync_copy(stg8, s_sp.at[pl.ds(r0s + i * stage, stage)])
      pltpu.sync_copy(stg1, c_sp.at[pl.ds(r0s + i * stage, stage)])

    @pl.loop(0, n_stage)
    def _(i):
      pltpu.sync_copy(y_hbm.at[pl.ds(r0s + i * stage, stage)], stg8)
      pltpu.sync_copy(stg8, y_sp.at[pl.ds(r0s + i * stage, stage)])
      pltpu.sync_copy(dinv_hbm.at[pl.ds(r0s + i * stage, stage)], stg1)
      pltpu.sync_copy(stg1, dinv_sp.at[pl.ds(r0s + i * stage, stage)])

    plsc.subcore_barrier()
    row_base = wid * rows_pt

    @pl.loop(0, rows_pt // rb)
    def _(b):
      pltpu.sync_copy(src_hbm.at[pl.ds(row_base + b * rb, rb)], sidx)
      pltpu.sync_copy(dst_hbm.at[pl.ds(row_base + b * rb, rb)], didx)
      gd = [pltpu.async_copy(y_sp.at[sidx.at[r]], ybufs.at[r], gsem)
            for r in range(rb)]
      dd = [pltpu.async_copy(dinv_sp.at[didx.at[r]], cvals.at[r], dsem)
            for r in range(rb)]
      for d in gd + dd:
        d.wait()
      sd = [pltpu.async_copy(ybufs.at[r], s_sp.at[didx.at[r]], ssem, add=True)
            for r in range(rb)]
      cd = [pltpu.async_copy(cvals.at[r], c_sp.at[sidx.at[r]], csem, add=True)
            for r in range(rb)]
      for d in sd + cd:
        d.wait()

    plsc.subcore_barrier()

    @pl.loop(0, n_stage)
    def _(i):
      pltpu.sync_copy(s_sp.at[pl.ds(r0s + i * stage, stage)], stg8)
      pltpu.sync_copy(stg8, s_out.at[pl.ds(c * n_pad + r0s + i * stage,
                                           stage)])
      pltpu.sync_copy(c_sp.at[pl.ds(r0s + i * stage, stage)], stg1)
      pltpu.sync_copy(stg1, c_out.at[pl.ds(c * n_pad + r0s + i * stage,
                                           stage)])

  return k


def _norm_tc(n_pad, n, blk):
  nb = n_pad // blk

  def body(p0_ref, p1_ref, x_ref, dinv_ref, y_ref):
    i = pl.program_id(0)
    deg = p0_ref[...] + p1_ref[...] + 1.0
    dinv = lax.rsqrt(deg)
    rid = lax.broadcasted_iota(jnp.int32, (blk, 1), 0) + i * blk
    dinv = jnp.where(rid < n, dinv, 0.0)
    dinv_ref[...] = dinv
    y_ref[...] = dinv * x_ref[...]

  return pl.pallas_call(
      body,
      grid=(nb,),
      in_specs=[
          pl.BlockSpec((blk, 1), lambda i: (i, 0)),
          pl.BlockSpec((blk, 1), lambda i: (i, 0)),
          pl.BlockSpec((blk, 8), lambda i: (i, 0)),
      ],
      out_specs=[
          pl.BlockSpec((blk, 1), lambda i: (i, 0)),
          pl.BlockSpec((blk, 8), lambda i: (i, 0)),
      ],
      out_shape=(jax.ShapeDtypeStruct((n_pad, 1), jnp.float32),
                 jax.ShapeDtypeStruct((n_pad, 8), jnp.float32)),
  )


def _fuse_tc(n_pad, n, blk):
  nb = n_pad // blk

  def body(s0_ref, s1_ref, c0_ref, c1_ref, dinv_ref, x_ref,
           w1_ref, b1_ref, w2_ref, b2_ref, st_ref, ws_ref, bs_ref,
           wc1_ref, wc2_ref, bc_ref, out_ref, acc):
    i = pl.program_id(0)

    @pl.when(i == 0)
    def _():
      acc[...] = jnp.zeros_like(acc)

    dv = dinv_ref[...]
    a1 = dv * (s0_ref[...] + s1_ref[...]) + dv * dv * x_ref[...]
    h1 = jnp.maximum(
        jnp.dot(a1, w1_ref[...], preferred_element_type=jnp.float32)
        + b1_ref[...], 0.0)
    w = dv * (c0_ref[...] + c1_ref[...] + dv) * (1.0 / n)
    acc[...] += jnp.sum(w * h1, axis=0, keepdims=True)

    @pl.when(i == nb - 1)
    def _():
      pooled = jnp.dot(acc[...], w2_ref[...],
                       preferred_element_type=jnp.float32) + b2_ref[...]
      sf = jnp.maximum(
          jnp.dot(st_ref[...], ws_ref[...],
                  preferred_element_type=jnp.float32) + bs_ref[...], 0.0)
      out_ref[...] = (
          jnp.dot(pooled, wc1_ref[...], preferred_element_type=jnp.float32)
          + jnp.dot(sf, wc2_ref[...], preferred_element_type=jnp.float32)
          + bc_ref[...])

  full = lambda shape: pl.BlockSpec(shape, lambda i: tuple(0 for _ in shape))
  return pl.pallas_call(
      body,
      grid=(nb,),
      in_specs=[
          pl.BlockSpec((blk, 8), lambda i: (i, 0)),
          pl.BlockSpec((blk, 8), lambda i: (i, 0)),
          pl.BlockSpec((blk, 1), lambda i: (i, 0)),
          pl.BlockSpec((blk, 1), lambda i: (i, 0)),
          pl.BlockSpec((blk, 1), lambda i: (i, 0)),
          pl.BlockSpec((blk, 8), lambda i: (i, 0)),
          full((8, 64)), full((1, 64)),
          full((64, 64)), full((1, 64)),
          full((1, 8)), full((8, 64)), full((1, 64)),
          full((64, 2)), full((64, 2)), full((1, 2)),
      ],
      out_specs=pl.BlockSpec((1, 2), lambda i: (0, 0)),
      out_shape=jax.ShapeDtypeStruct((1, 2), jnp.float32),
      scratch_shapes=[pltpu.VMEM((1, 64), jnp.float32)],
  )


def kernel(x, edge_index, state, W1, b1, W2, b2, Ws, bs, Wc, bc):
  n, _ = x.shape
  e = edge_index.shape[1]
  n_pad = ((n + 1 + LANES - 1) // LANES) * LANES
  rows_pt8 = (e + NW * LANES - 1) // (NW * LANES)
  rows_pt = ((rows_pt8 + 7) // 8) * 8
  e_pad = NW * rows_pt * LANES
  rb = 8

  sent = (n + (jnp.arange(e_pad - e, dtype=jnp.int32) % (n_pad - n)))
  src2d = jnp.concatenate([edge_index[0], sent]).reshape(-1, LANES)
  dst2d = jnp.concatenate([edge_index[1], sent]).reshape(-1, LANES)
  x_pad = jnp.pad(x, ((0, n_pad - n), (0, 0)))
  zeros8 = jnp.zeros((_stage_of(n_pad // NS), 8), jnp.float32)

  degp = _deg_kernel(n_pad, rows_pt, rb)(dst2d).reshape(NC, n_pad)
  p0 = degp[0].reshape(n_pad, 1)
  p1 = degp[1].reshape(n_pad, 1)

  blk = n_pad // 16
  dinv2d, y = _norm_tc(n_pad, n, blk)(p0, p1, x_pad)

  sp, cp = _main_kernel(n_pad, rows_pt, rb)(
      src2d, dst2d, y, dinv2d.reshape(n_pad), zeros8)
  sp = sp.reshape(NC, n_pad, 8)
  cp = cp.reshape(NC, n_pad)

  out = _fuse_tc(n_pad, n, blk)(
      sp[0], sp[1], cp[0].reshape(n_pad, 1), cp[1].reshape(n_pad, 1),
      dinv2d, x_pad,
      W1, b1.reshape(1, -1), W2, b2.reshape(1, -1),
      state, Ws, bs.reshape(1, -1), Wc[:64], Wc[64:], bc.reshape(1, -1))
  return out


# speedup vs baseline: 130.6812x; 1.5594x over previous
import functools

import jax
import jax.numpy as jnp
from jax import lax
from jax.experimental import pallas as pl
from jax.experimental.pallas import tpu as pltpu
from jax.experimental.pallas import tpu_sc as plsc

NC = 2
NS = 16
NW = NC * NS
LANES = 128


def _mesh():
  return plsc.VectorSubcoreMesh(core_axis_name="c", subcore_axis_name="s")


def _stage_of(slc, cap=512):
  return next(s for s in range(cap, 7, -8) if slc % s == 0)


def _deg_kernel(n_pad, rows_pt, rb):
  slc = n_pad // NS
  stage = _stage_of(slc)
  n_stage = slc // stage

  @functools.partial(
      pl.kernel,
      out_type=jax.ShapeDtypeStruct((NC * n_pad,), jnp.float32),
      mesh=_mesh(),
      compiler_params=pltpu.CompilerParams(use_tc_tiling_on_sc=False),
      scratch_types=[
          pltpu.VMEM_SHARED((n_pad,), jnp.float32),
          pltpu.VMEM((rb, LANES), jnp.int32),
          pltpu.VMEM((LANES,), jnp.float32),
          pltpu.VMEM((stage,), jnp.float32),
          pltpu.SemaphoreType.DMA,
      ],
  )
  def k(dst_hbm, out_hbm, deg_sp, idx_v, ones_v, stg1, ssem):
    c = lax.axis_index("c")
    s = lax.axis_index("s")
    wid = c * NS + s
    r0s = s * slc

    @pl.loop(0, stage // 16)
    def _(i):
      stg1[pl.ds(i * 16, 16)] = jnp.zeros((16,), jnp.float32)

    @pl.loop(0, n_stage)
    def _(i):
      pltpu.sync_copy(stg1, deg_sp.at[pl.ds(r0s + i * stage, stage)])

    for j in range(LANES // 16):
      ones_v[pl.ds(j * 16, 16)] = jnp.full((16,), 1.0, jnp.float32)
    plsc.subcore_barrier()
    row_base = wid * rows_pt

    @pl.loop(0, rows_pt // rb)
    def _(b):
      pltpu.sync_copy(dst_hbm.at[pl.ds(row_base + b * rb, rb)], idx_v)
      descs = [pltpu.async_copy(ones_v, deg_sp.at[idx_v.at[r]], ssem,
                                add=True) for r in range(rb)]
      for d in descs:
        d.wait()

    plsc.subcore_barrier()

    @pl.loop(0, n_stage)
    def _(i):
      pltpu.sync_copy(deg_sp.at[pl.ds(r0s + i * stage, stage)], stg1)
      pltpu.sync_copy(stg1, out_hbm.at[pl.ds(c * n_pad + r0s + i * stage,
                                             stage)])

  return k


def _main_kernel(n_pad, rows_pt, rb):
  slc = n_pad // NS
  stage = _stage_of(slc)
  n_stage = slc // stage

  @functools.partial(
      pl.kernel,
      out_type=(jax.ShapeDtypeStruct((NC * n_pad, 8), jnp.float32),
                jax.ShapeDtypeStruct((NC * n_pad,), jnp.float32)),
      mesh=_mesh(),
      compiler_params=pltpu.CompilerParams(use_tc_tiling_on_sc=False),
      scratch_types=[
          pltpu.VMEM_SHARED((n_pad, 8), jnp.float32),
          pltpu.VMEM_SHARED((n_pad, 8), jnp.float32),
          pltpu.VMEM_SHARED((n_pad,), jnp.float32),
          pltpu.VMEM_SHARED((n_pad,), jnp.float32),
          pltpu.VMEM((rb, LANES), jnp.int32),
          pltpu.VMEM((rb, LANES), jnp.int32),
          pltpu.VMEM((rb, LANES, 8), jnp.float32),
          pltpu.VMEM((rb, LANES), jnp.float32),
          pltpu.VMEM((stage, 8), jnp.float32),
          pltpu.VMEM((stage,), jnp.float32),
          pltpu.SemaphoreType.DMA,
          pltpu.SemaphoreType.DMA,
          pltpu.SemaphoreType.DMA,
          pltpu.SemaphoreType.DMA,
      ],
  )
  def k(src_hbm, dst_hbm, y_hbm, dinv_hbm, z8_hbm,
        s_out, c_out, y_sp, s_sp, c_sp, dinv_sp, sidx, didx, ybufs, cvals,
        stg8, stg1, gsem, dsem, ssem, csem):
    c = lax.axis_index("c")
    s = lax.axis_index("s")
    wid = c * NS + s
    r0s = s * slc

    @pl.loop(0, stage // 16)
    def _(i):
      stg1[pl.ds(i * 16, 16)] = jnp.zeros((16,), jnp.float32)

    pltpu.sync_copy(z8_hbm, stg8)

    @pl.loop(0, n_stage)
    def _(i):
      pltpu.sync_copy(stg8, s_sp.at[pl.ds(r0s + i * stage, stage)])
      pltpu.sync_copy(stg1, c_sp.at[pl.ds(r0s + i * stage, stage)])

    @pl.loop(0, n_stage)
    def _(i):
      pltpu.sync_copy(y_hbm.at[pl.ds(r0s + i * stage, stage)], stg8)
      pltpu.sync_copy(stg8, y_sp.at[pl.ds(r0s + i * stage, stage)])
      pltpu.sync_copy(dinv_hbm.at[pl.ds(r0s + i * stage, stage)], stg1)
      pltpu.sync_copy(stg1, dinv_sp.at[pl.ds(r0s + i * stage, stage)])

    plsc.subcore_barrier()
    row_base = wid * rows_pt

    @pl.loop(0, rows_pt // rb)
    def _(b):
      pltpu.sync_copy(src_hbm.at[pl.ds(row_base + b * rb, rb)], sidx)
      pltpu.sync_copy(dst_hbm.at[pl.ds(row_base + b * rb, rb)], didx)
      gd = [pltpu.async_copy(y_sp.at[sidx.at[r]], ybufs.at[r], gsem)
            for r in range(rb)]
      dd = [pltpu.async_copy(dinv_sp.at[didx.at[r]], cvals.at[r], dsem)
            for r in range(rb)]
      for d in gd + dd:
        d.wait()
      sd = [pltpu.async_copy(ybufs.at[r], s_sp.at[didx.at[r]], ssem, add=True)
            for r in range(rb)]
      cd = [pltpu.async_copy(cvals.at[r], c_sp.at[sidx.at[r]], csem, add=True)
            for r in range(rb)]
      for d in sd + cd:
        d.wait()

    plsc.subcore_barrier()

    @pl.loop(0, n_stage)
    def _(i):
      pltpu.sync_copy(s_sp.at[pl.ds(r0s + i * stage, stage)], stg8)
      pltpu.sync_copy(stg8, s_out.at[pl.ds(c * n_pad + r0s + i * stage,
                                           stage)])
      pltpu.sync_copy(c_sp.at[pl.ds(r0s + i * stage, stage)], stg1)
      pltpu.sync_copy(stg1, c_out.at[pl.ds(c * n_pad + r0s + i * stage,
                                           stage)])

  return k


def _norm_tc(n_pad, n, blk):
  nb = n_pad // blk

  def body(p0_ref, p1_ref, x_ref, dinv_ref, y_ref):
    i = pl.program_id(0)
    deg = p0_ref[...] + p1_ref[...] + 1.0
    dinv = lax.rsqrt(deg)
    rid = lax.broadcasted_iota(jnp.int32, (blk,), 0) + i * blk
    dinv = jnp.where(rid < n, dinv, 0.0)
    dinv_ref[...] = dinv
    y_ref[...] = dinv.reshape(blk, 1) * x_ref[...]

  return pl.pallas_call(
      body,
      grid=(nb,),
      in_specs=[
          pl.BlockSpec((blk,), lambda i: (i,)),
          pl.BlockSpec((blk,), lambda i: (i + nb,)),
          pl.BlockSpec((blk, 8), lambda i: (i, 0)),
      ],
      out_specs=[
          pl.BlockSpec((blk,), lambda i: (i,)),
          pl.BlockSpec((blk, 8), lambda i: (i, 0)),
      ],
      out_shape=(jax.ShapeDtypeStruct((n_pad,), jnp.float32),
                 jax.ShapeDtypeStruct((n_pad, 8), jnp.float32)),
  )


def _fuse_tc(n_pad, n, blk):
  nb = n_pad // blk

  def body(s0_ref, s1_ref, x_ref, dv_ref, c0_ref, c1_ref,
           w1_ref, b1_ref, w2_ref, b2_ref, st_ref, ws_ref, bs_ref,
           wc1_ref, wc2_ref, bc_ref, out_ref, acc):
    i = pl.program_id(0)

    @pl.when(i == 0)
    def _():
      acc[...] = jnp.zeros_like(acc)

    dv = dv_ref[...]
    dvc = dv.reshape(blk, 1)
    a1 = dvc * (s0_ref[...] + s1_ref[...] + dvc * x_ref[...])
    h1 = jnp.maximum(
        jnp.dot(a1, w1_ref[...], preferred_element_type=jnp.float32)
        + b1_ref[...], 0.0)
    w = dv * (c0_ref[...] + c1_ref[...] + dv) * (1.0 / n)
    acc[...] += jnp.dot(w.reshape(1, blk), h1,
                        preferred_element_type=jnp.float32)

    @pl.when(i == nb - 1)
    def _():
      pooled = jnp.dot(acc[...], w2_ref[...],
                       preferred_element_type=jnp.float32) + b2_ref[...]
      sf = jnp.maximum(
          jnp.dot(st_ref[...], ws_ref[...],
                  preferred_element_type=jnp.float32) + bs_ref[...], 0.0)
      out_ref[...] = (
          jnp.dot(pooled, wc1_ref[...], preferred_element_type=jnp.float32)
          + jnp.dot(sf, wc2_ref[...], preferred_element_type=jnp.float32)
          + bc_ref[...])

  full = lambda shape: pl.BlockSpec(shape, lambda i: tuple(0 for _ in shape))
  return pl.pallas_call(
      body,
      grid=(nb,),
      in_specs=[
          pl.BlockSpec((blk, 8), lambda i: (i, 0)),
          pl.BlockSpec((blk, 8), lambda i: (i + nb, 0)),
          pl.BlockSpec((blk, 8), lambda i: (i, 0)),
          pl.BlockSpec((blk,), lambda i: (i,)),
          pl.BlockSpec((blk,), lambda i: (i,)),
          pl.BlockSpec((blk,), lambda i: (i + nb,)),
          full((8, 64)), full((1, 64)),
          full((64, 64)), full((1, 64)),
          full((1, 8)), full((8, 64)), full((1, 64)),
          full((64, 2)), full((64, 2)), full((1, 2)),
      ],
      out_specs=pl.BlockSpec((1, 2), lambda i: (0, 0)),
      out_shape=jax.ShapeDtypeStruct((1, 2), jnp.float32),
      scratch_shapes=[pltpu.VMEM((1, 64), jnp.float32)],
  )


def kernel(x, edge_index, state, W1, b1, W2, b2, Ws, bs, Wc, bc):
  n, _ = x.shape
  e = edge_index.shape[1]
  n_pad = ((n + 1 + 1023) // 1024) * 1024
  rows_pt8 = (e + NW * LANES - 1) // (NW * LANES)
  rows_pt = ((rows_pt8 + 7) // 8) * 8
  e_pad = NW * rows_pt * LANES
  rb = 8

  sent = (n + (jnp.arange(e_pad - e, dtype=jnp.int32) % (n_pad - n)))
  src2d = jnp.concatenate([edge_index[0], sent]).reshape(-1, LANES)
  dst2d = jnp.concatenate([edge_index[1], sent]).reshape(-1, LANES)
  x_pad = jnp.pad(x, ((0, n_pad - n), (0, 0)))
  zeros8 = jnp.zeros((_stage_of(n_pad // NS), 8), jnp.float32)

  degp = _deg_kernel(n_pad, rows_pt, rb)(dst2d)

  blk = next(n_pad // nb for nb in (14, 16, 8, 4, 2, 1)
             if n_pad % (nb * 1024) == 0)
  dinv, y = _norm_tc(n_pad, n, blk)(degp, degp, x_pad)

  sp, cp = _main_kernel(n_pad, rows_pt, rb)(src2d, dst2d, y, dinv, zeros8)

  out = _fuse_tc(n_pad, n, blk)(
      sp, sp, x_pad, dinv, cp, cp,
      W1, b1.reshape(1, -1), W2, b2.reshape(1, -1),
      state, Ws, bs.reshape(1, -1), Wc[:64], Wc[64:], bc.reshape(1, -1))
  return out
